# Initial kernel scaffold; baseline (speedup 1.0000x reference)
import jax
import jax.numpy as jnp
from jax.experimental import pallas as pl


def kernel(x, pos_table):
    raise NotImplementedError("write your pallas kernel here")


# speedup vs baseline: 2.3837x; 2.3837x over previous
import jax
import jax.numpy as jnp
from jax.experimental import pallas as pl

MAX_LEN = 4096
EMB = 128
BATCH = 4

S_BLK = 1024


def _add_pos_kernel(x_ref, pos_ref, out_ref):
    out_ref[...] = x_ref[...] + pos_ref[...][None, :, :]


def kernel(x, pos_table):
    grid = (MAX_LEN // S_BLK, BATCH)
    return pl.pallas_call(
        _add_pos_kernel,
        grid=grid,
        in_specs=[
            pl.BlockSpec((1, S_BLK, EMB), lambda i, b: (b, i, 0)),
            pl.BlockSpec((S_BLK, EMB), lambda i, b: (i, 0)),
        ],
        out_specs=pl.BlockSpec((1, S_BLK, EMB), lambda i, b: (b, i, 0)),
        out_shape=jax.ShapeDtypeStruct((BATCH, MAX_LEN, EMB), jnp.float32),
    )(x, pos_table)


# speedup vs baseline: 3.3124x; 1.3896x over previous
import jax
import jax.numpy as jnp
from jax.experimental import pallas as pl

MAX_LEN = 4096
EMB = 128
BATCH = 4

S_BLK = 2048


def _add_pos_kernel(x_ref, pos_ref, out_ref):
    out_ref[...] = x_ref[...] + pos_ref[...][None, :, :]


def kernel(x, pos_table):
    grid = (MAX_LEN // S_BLK, BATCH)
    return pl.pallas_call(
        _add_pos_kernel,
        grid=grid,
        in_specs=[
            pl.BlockSpec((1, S_BLK, EMB), lambda i, b: (b, i, 0)),
            pl.BlockSpec((S_BLK, EMB), lambda i, b: (i, 0)),
        ],
        out_specs=pl.BlockSpec((1, S_BLK, EMB), lambda i, b: (b, i, 0)),
        out_shape=jax.ShapeDtypeStruct((BATCH, MAX_LEN, EMB), jnp.float32),
    )(x, pos_table)


# speedup vs baseline: 4.1257x; 1.2455x over previous
import jax
import jax.numpy as jnp
from jax.experimental import pallas as pl

MAX_LEN = 4096
EMB = 128
BATCH = 4

S_BLK = 4096


def _add_pos_kernel(x_ref, pos_ref, out_ref):
    out_ref[...] = x_ref[...] + pos_ref[...][None, :, :]


def kernel(x, pos_table):
    grid = (MAX_LEN // S_BLK, BATCH)
    return pl.pallas_call(
        _add_pos_kernel,
        grid=grid,
        in_specs=[
            pl.BlockSpec((1, S_BLK, EMB), lambda i, b: (b, i, 0)),
            pl.BlockSpec((S_BLK, EMB), lambda i, b: (i, 0)),
        ],
        out_specs=pl.BlockSpec((1, S_BLK, EMB), lambda i, b: (b, i, 0)),
        out_shape=jax.ShapeDtypeStruct((BATCH, MAX_LEN, EMB), jnp.float32),
    )(x, pos_table)


# speedup vs baseline: 5.0076x; 1.2138x over previous
import jax
import jax.numpy as jnp
from jax.experimental import pallas as pl

MAX_LEN = 4096
EMB = 128
BATCH = 4

S_BLK = 4096


def _add_pos_kernel(x_ref, pos_ref, out_ref):
    out_ref[...] = x_ref[...] + pos_ref[...][None, :, :]


B_BLK = 2


def kernel(x, pos_table):
    grid = (MAX_LEN // S_BLK, BATCH // B_BLK)
    return pl.pallas_call(
        _add_pos_kernel,
        grid=grid,
        in_specs=[
            pl.BlockSpec((B_BLK, S_BLK, EMB), lambda i, b: (b, i, 0)),
            pl.BlockSpec((S_BLK, EMB), lambda i, b: (i, 0)),
        ],
        out_specs=pl.BlockSpec((B_BLK, S_BLK, EMB), lambda i, b: (b, i, 0)),
        out_shape=jax.ShapeDtypeStruct((BATCH, MAX_LEN, EMB), jnp.float32),
    )(x, pos_table)
